# Initial kernel scaffold; baseline (speedup 1.0000x reference)
import jax
import jax.numpy as jnp
from jax.experimental import pallas as pl


def kernel(tokens, alt_embedding, W1, b1, W2, b2):
    raise NotImplementedError("write your pallas kernel here")


# speedup vs baseline: 2.8026x; 2.8026x over previous
import functools

import jax
import jax.numpy as jnp
from jax.experimental import pallas as pl
from jax.experimental.pallas import tpu as pltpu

D_MODEL = 2048
ALT_DIM = 32
NUM_EXPERTS = 16
TOP_K = 2


def _router_kernel(tokens_ref, alt_ref, w1t_ref, w1a_ref, b1_ref, w2_ref,
                   b2_ref, gates_ref, idx_ref, loss_ref,
                   f_acc, p_acc, *, n_tokens_total, grid_b, grid_n):
    b = pl.program_id(0)
    n = pl.program_id(1)
    is_first = jnp.logical_and(b == 0, n == 0)
    is_last = jnp.logical_and(b == grid_b - 1, n == grid_n - 1)

    @pl.when(is_first)
    def _init():
        f_acc[...] = jnp.zeros_like(f_acc)
        p_acc[...] = jnp.zeros_like(p_acc)

    x = tokens_ref[0]
    alt = alt_ref[0]

    acc = jnp.dot(x.astype(jnp.bfloat16), w1t_ref[...].astype(jnp.bfloat16),
                  preferred_element_type=jnp.float32)
    alt_h = jnp.dot(alt.astype(jnp.bfloat16), w1a_ref[...].astype(jnp.bfloat16),
                    preferred_element_type=jnp.float32)
    pre = acc + alt_h + b1_ref[...]
    h = 0.5 * pre * (1.0 + jax.lax.erf(pre * (2.0 ** -0.5)))

    logits = jnp.dot(h.astype(jnp.bfloat16), w2_ref[...].astype(jnp.bfloat16),
                     preferred_element_type=jnp.float32) + b2_ref[...]

    col = jax.lax.broadcasted_iota(jnp.int32, logits.shape, 1)
    m1 = jnp.max(logits, axis=1, keepdims=True)
    i1 = jnp.argmax(logits, axis=1).astype(jnp.int32)
    masked = jnp.where(col == i1[:, None], -jnp.inf, logits)
    m2 = jnp.max(masked, axis=1, keepdims=True)
    i2 = jnp.argmax(masked, axis=1).astype(jnp.int32)

    e = jnp.exp(m2 - m1)
    g1 = 1.0 / (1.0 + e)
    gates_ref[0] = jnp.concatenate([g1, 1.0 - g1], axis=1)
    idx_ref[0] = jnp.stack([i1, i2], axis=1)

    probs = jnp.exp(logits - m1)
    probs = probs / jnp.sum(probs, axis=1, keepdims=True)
    onehot1 = (col == i1[:, None]).astype(jnp.float32)
    f_acc[...] += jnp.sum(onehot1, axis=0, keepdims=True)
    p_acc[...] += jnp.sum(probs, axis=0, keepdims=True)

    @pl.when(is_last)
    def _finish():
        inv = 1.0 / n_tokens_total
        loss_ref[...] = (NUM_EXPERTS * jnp.sum((f_acc[...] * inv) *
                                               (p_acc[...] * inv))).reshape(1, 1)


def kernel(tokens, alt_embedding, W1, b1, W2, b2):
    B, N, D = tokens.shape
    BN = 512
    grid_b, grid_n = B, N // BN

    W1t = W1[:D]
    W1a = W1[D:]
    alt3 = alt_embedding.reshape(B, 1, ALT_DIM)
    b1r = b1.reshape(1, -1)
    b2r = b2.reshape(1, -1)

    grid_spec = dict(
        grid=(grid_b, grid_n),
        in_specs=[
            pl.BlockSpec((1, BN, D), lambda b, n: (b, n, 0)),
            pl.BlockSpec((1, 1, ALT_DIM), lambda b, n: (b, 0, 0)),
            pl.BlockSpec((D, W1.shape[1]), lambda b, n: (0, 0)),
            pl.BlockSpec((ALT_DIM, W1.shape[1]), lambda b, n: (0, 0)),
            pl.BlockSpec((1, b1.shape[0]), lambda b, n: (0, 0)),
            pl.BlockSpec(W2.shape, lambda b, n: (0, 0)),
            pl.BlockSpec((1, NUM_EXPERTS), lambda b, n: (0, 0)),
        ],
        out_specs=[
            pl.BlockSpec((1, BN, TOP_K), lambda b, n: (b, n, 0)),
            pl.BlockSpec((1, BN, TOP_K), lambda b, n: (b, n, 0)),
            pl.BlockSpec((1, 1), lambda b, n: (0, 0)),
        ],
    )

    gates, idx, loss = pl.pallas_call(
        functools.partial(_router_kernel, n_tokens_total=float(B * N),
                          grid_b=grid_b, grid_n=grid_n),
        **grid_spec,
        out_shape=[
            jax.ShapeDtypeStruct((B, N, TOP_K), jnp.float32),
            jax.ShapeDtypeStruct((B, N, TOP_K), jnp.int32),
            jax.ShapeDtypeStruct((1, 1), jnp.float32),
        ],
        scratch_shapes=[
            pltpu.VMEM((1, NUM_EXPERTS), jnp.float32),
            pltpu.VMEM((1, NUM_EXPERTS), jnp.float32),
        ],
    )(tokens, alt3, W1t, W1a, b1r, W2, b2r)

    return gates, idx, loss[0, 0]


# speedup vs baseline: 3.0160x; 1.0761x over previous
import functools

import jax
import jax.numpy as jnp
from jax.experimental import pallas as pl
from jax.experimental.pallas import tpu as pltpu

D_MODEL = 2048
ALT_DIM = 32
NUM_EXPERTS = 16
TOP_K = 2


def _router_kernel(tokens_ref, alt_ref, w1t_ref, w1a_ref, b1_ref, w2_ref,
                   b2_ref, gates_ref, idx_ref, loss_ref,
                   f_acc, p_acc, *, n_tokens_total, grid_b, grid_n):
    b = pl.program_id(0)
    n = pl.program_id(1)
    is_first = jnp.logical_and(b == 0, n == 0)
    is_last = jnp.logical_and(b == grid_b - 1, n == grid_n - 1)

    @pl.when(is_first)
    def _init():
        f_acc[...] = jnp.zeros_like(f_acc)
        p_acc[...] = jnp.zeros_like(p_acc)

    x = tokens_ref[0]
    alt = alt_ref[0]

    acc = jnp.dot(x.astype(jnp.bfloat16), w1t_ref[...],
                  preferred_element_type=jnp.float32)
    alt_h = jnp.dot(alt.astype(jnp.bfloat16), w1a_ref[...],
                    preferred_element_type=jnp.float32)
    pre = acc + alt_h + b1_ref[...]
    h = 0.5 * pre * (1.0 + jax.lax.erf(pre * (2.0 ** -0.5)))

    logits = jnp.dot(h.astype(jnp.bfloat16), w2_ref[...],
                     preferred_element_type=jnp.float32) + b2_ref[...]

    col = jax.lax.broadcasted_iota(jnp.int32, logits.shape, 1)
    m1 = jnp.max(logits, axis=1, keepdims=True)
    i1 = jnp.argmax(logits, axis=1).astype(jnp.int32)
    masked = jnp.where(col == i1[:, None], -jnp.inf, logits)
    m2 = jnp.max(masked, axis=1, keepdims=True)
    i2 = jnp.argmax(masked, axis=1).astype(jnp.int32)

    e = jnp.exp(m2 - m1)
    g1 = 1.0 / (1.0 + e)
    gates_ref[0] = jnp.concatenate([g1, 1.0 - g1], axis=1)
    idx_ref[0] = jnp.stack([i1, i2], axis=1)

    probs = jnp.exp(logits - m1)
    probs = probs / jnp.sum(probs, axis=1, keepdims=True)
    onehot1 = (col == i1[:, None]).astype(jnp.float32)
    f_acc[...] += jnp.sum(onehot1, axis=0, keepdims=True)
    p_acc[...] += jnp.sum(probs, axis=0, keepdims=True)

    @pl.when(is_last)
    def _finish():
        inv = 1.0 / n_tokens_total
        loss_ref[...] = (NUM_EXPERTS * jnp.sum((f_acc[...] * inv) *
                                               (p_acc[...] * inv))).reshape(1, 1)


def kernel(tokens, alt_embedding, W1, b1, W2, b2):
    B, N, D = tokens.shape
    BN = 512
    grid_b, grid_n = B, N // BN

    W1t = W1[:D].astype(jnp.bfloat16)
    W1a = W1[D:].astype(jnp.bfloat16)
    W2b = W2.astype(jnp.bfloat16)
    alt3 = alt_embedding.reshape(B, 1, ALT_DIM)
    b1r = b1.reshape(1, -1)
    b2r = b2.reshape(1, -1)

    grid_spec = dict(
        grid=(grid_b, grid_n),
        in_specs=[
            pl.BlockSpec((1, BN, D), lambda b, n: (b, n, 0)),
            pl.BlockSpec((1, 1, ALT_DIM), lambda b, n: (b, 0, 0)),
            pl.BlockSpec((D, W1.shape[1]), lambda b, n: (0, 0)),
            pl.BlockSpec((ALT_DIM, W1.shape[1]), lambda b, n: (0, 0)),
            pl.BlockSpec((1, b1.shape[0]), lambda b, n: (0, 0)),
            pl.BlockSpec(W2b.shape, lambda b, n: (0, 0)),
            pl.BlockSpec((1, NUM_EXPERTS), lambda b, n: (0, 0)),
        ],
        out_specs=[
            pl.BlockSpec((1, BN, TOP_K), lambda b, n: (b, n, 0)),
            pl.BlockSpec((1, BN, TOP_K), lambda b, n: (b, n, 0)),
            pl.BlockSpec((1, 1), lambda b, n: (0, 0)),
        ],
    )

    gates, idx, loss = pl.pallas_call(
        functools.partial(_router_kernel, n_tokens_total=float(B * N),
                          grid_b=grid_b, grid_n=grid_n),
        **grid_spec,
        out_shape=[
            jax.ShapeDtypeStruct((B, N, TOP_K), jnp.float32),
            jax.ShapeDtypeStruct((B, N, TOP_K), jnp.int32),
            jax.ShapeDtypeStruct((1, 1), jnp.float32),
        ],
        scratch_shapes=[
            pltpu.VMEM((1, NUM_EXPERTS), jnp.float32),
            pltpu.VMEM((1, NUM_EXPERTS), jnp.float32),
        ],
    )(tokens, alt3, W1t, W1a, b1r, W2b, b2r)

    return gates, idx, loss[0, 0]
